# Initial kernel scaffold; baseline (speedup 1.0000x reference)
import jax
import jax.numpy as jnp
from jax.experimental import pallas as pl


def kernel(edge_attr_cat, edge_attr_num, emb_acc, emb_trans, emb_season, W_num, b_num, W_out, b_out):
    raise NotImplementedError("write your pallas kernel here")


# speedup vs baseline: 7.9493x; 7.9493x over previous
import jax
import jax.numpy as jnp
from jax.experimental import pallas as pl

E = 160000
BLOCK = 1280
NB = E // BLOCK


def _fold_kernel(a_ref, w_ref, bout_ref, m_ref):
    acc = jnp.zeros((128, 512), dtype=jnp.float32)
    for k in range(4):
        acc = acc + jax.lax.dot_general(
            a_ref[k], w_ref[k],
            dimension_numbers=(((1,), (0,)), ((), ())),
            preferred_element_type=jnp.float32,
        )
    row = jax.lax.broadcasted_iota(jnp.int32, (128, 512), 0)
    m_ref[...] = acc + jnp.where(row == 90, bout_ref[...], 0.0)


def _main_kernel(i0_ref, i1_ref, i2_ref, xnt_ref, m_ref, out_ref):
    b = out_ref.shape[0]
    l = jax.lax.broadcasted_iota(jnp.int32, (112, b), 0)
    i0 = i0_ref[0]
    i1 = i1_ref[0]
    i2 = i2_ref[0]
    oh = ((l == i0) | (l == i1 + 50) | (l == i2 + 70) | (l == 74)).astype(
        jnp.float32
    )
    lhs_t = jnp.concatenate([xnt_ref[...], oh], axis=0)
    out_ref[...] = jax.lax.dot_general(
        lhs_t, m_ref[...],
        dimension_numbers=(((0,), (0,)), ((), ())),
        preferred_element_type=jnp.float32,
    )


def kernel(edge_attr_cat, edge_attr_num, emb_acc, emb_trans, emb_season,
           W_num, b_num, W_out, b_out):
    f32 = jnp.float32
    z = lambda n: jnp.zeros((n, 128), dtype=f32)
    a0 = jnp.concatenate([z(16), emb_acc.astype(f32), z(62)], axis=0)
    a1 = jnp.concatenate([z(66), emb_trans.astype(f32), z(42)], axis=0)
    a2 = jnp.concatenate([z(86), emb_season.astype(f32), z(38)], axis=0)
    a3 = jnp.concatenate(
        [W_num.astype(f32), z(74), b_num.astype(f32)[None, :], z(37)], axis=0
    )
    astack = jnp.stack([a0, a1, a2, a3], axis=0)
    w_blocks = W_out.astype(f32).reshape(4, 128, 512)

    m = pl.pallas_call(
        _fold_kernel,
        out_shape=jax.ShapeDtypeStruct((128, 512), f32),
    )(astack, w_blocks, b_out.astype(f32)[None, :])

    idx = edge_attr_cat.astype(jnp.int32)
    i0 = idx[:, 0].reshape(NB, 1, BLOCK)
    i1 = idx[:, 1].reshape(NB, 1, BLOCK)
    i2 = idx[:, 2].reshape(NB, 1, BLOCK)
    xnum_t = edge_attr_num.astype(f32).T

    out = pl.pallas_call(
        _main_kernel,
        grid=(NB,),
        in_specs=[
            pl.BlockSpec((1, 1, BLOCK), lambda i: (i, 0, 0)),
            pl.BlockSpec((1, 1, BLOCK), lambda i: (i, 0, 0)),
            pl.BlockSpec((1, 1, BLOCK), lambda i: (i, 0, 0)),
            pl.BlockSpec((16, BLOCK), lambda i: (0, i)),
            pl.BlockSpec((128, 512), lambda i: (0, 0)),
        ],
        out_specs=pl.BlockSpec((BLOCK, 512), lambda i: (i, 0)),
        out_shape=jax.ShapeDtypeStruct((E, 512), f32),
    )(i0, i1, i2, xnum_t, m)
    return out


# speedup vs baseline: 10.8394x; 1.3636x over previous
import jax
import jax.numpy as jnp
from jax.experimental import pallas as pl

E = 160000
BLOCK = 3200
NB = E // BLOCK


def _fold_kernel(a_ref, w_ref, bout_ref, m_ref):
    acc = jnp.zeros((128, 512), dtype=jnp.float32)
    for k in range(4):
        acc = acc + jax.lax.dot_general(
            a_ref[k], w_ref[k],
            dimension_numbers=(((1,), (0,)), ((), ())),
            preferred_element_type=jnp.float32,
        )
    row = jax.lax.broadcasted_iota(jnp.int32, (128, 512), 0)
    m_ref[...] = acc + jnp.where(row == 90, bout_ref[...], 0.0)


def _main_kernel(i0_ref, i1_ref, i2_ref, xnt_ref, m_ref, out_ref):
    b = out_ref.shape[0]
    l = jax.lax.broadcasted_iota(jnp.int32, (112, b), 0)
    i0 = i0_ref[0]
    i1 = i1_ref[0]
    i2 = i2_ref[0]
    oh = ((l == i0) | (l == i1 + 50) | (l == i2 + 70) | (l == 74)).astype(
        jnp.float32
    )
    lhs_t = jnp.concatenate([xnt_ref[...], oh], axis=0)
    out_ref[...] = jax.lax.dot_general(
        lhs_t, m_ref[...],
        dimension_numbers=(((0,), (0,)), ((), ())),
        preferred_element_type=jnp.float32,
    )


def kernel(edge_attr_cat, edge_attr_num, emb_acc, emb_trans, emb_season,
           W_num, b_num, W_out, b_out):
    f32 = jnp.float32
    z = lambda n: jnp.zeros((n, 128), dtype=f32)
    a0 = jnp.concatenate([z(16), emb_acc.astype(f32), z(62)], axis=0)
    a1 = jnp.concatenate([z(66), emb_trans.astype(f32), z(42)], axis=0)
    a2 = jnp.concatenate([z(86), emb_season.astype(f32), z(38)], axis=0)
    a3 = jnp.concatenate(
        [W_num.astype(f32), z(74), b_num.astype(f32)[None, :], z(37)], axis=0
    )
    astack = jnp.stack([a0, a1, a2, a3], axis=0)
    w_blocks = W_out.astype(f32).reshape(4, 128, 512)

    m = pl.pallas_call(
        _fold_kernel,
        out_shape=jax.ShapeDtypeStruct((128, 512), f32),
    )(astack, w_blocks, b_out.astype(f32)[None, :])

    idx = edge_attr_cat.astype(jnp.int32)
    i0 = idx[:, 0].reshape(NB, 1, BLOCK)
    i1 = idx[:, 1].reshape(NB, 1, BLOCK)
    i2 = idx[:, 2].reshape(NB, 1, BLOCK)
    xnum_t = edge_attr_num.astype(f32).T

    out = pl.pallas_call(
        _main_kernel,
        grid=(NB,),
        in_specs=[
            pl.BlockSpec((1, 1, BLOCK), lambda i: (i, 0, 0)),
            pl.BlockSpec((1, 1, BLOCK), lambda i: (i, 0, 0)),
            pl.BlockSpec((1, 1, BLOCK), lambda i: (i, 0, 0)),
            pl.BlockSpec((16, BLOCK), lambda i: (0, i)),
            pl.BlockSpec((128, 512), lambda i: (0, 0)),
        ],
        out_specs=pl.BlockSpec((BLOCK, 512), lambda i: (i, 0)),
        out_shape=jax.ShapeDtypeStruct((E, 512), f32),
    )(i0, i1, i2, xnum_t, m)
    return out


# speedup vs baseline: 11.0604x; 1.0204x over previous
import jax
import jax.numpy as jnp
from jax.experimental import pallas as pl

E = 160000
BLOCK = 6400
NB = E // BLOCK


def _fold_kernel(a_ref, w_ref, bout_ref, m_ref):
    acc = jnp.zeros((128, 512), dtype=jnp.float32)
    for k in range(4):
        acc = acc + jax.lax.dot_general(
            a_ref[k], w_ref[k],
            dimension_numbers=(((1,), (0,)), ((), ())),
            preferred_element_type=jnp.float32,
        )
    row = jax.lax.broadcasted_iota(jnp.int32, (128, 512), 0)
    m_ref[...] = acc + jnp.where(row == 90, bout_ref[...], 0.0)


def _main_kernel(i0_ref, i1_ref, i2_ref, xnt_ref, m_ref, out_ref):
    b = out_ref.shape[0]
    l = jax.lax.broadcasted_iota(jnp.int32, (112, b), 0)
    i0 = i0_ref[0]
    i1 = i1_ref[0]
    i2 = i2_ref[0]
    oh = ((l == i0) | (l == i1 + 50) | (l == i2 + 70) | (l == 74)).astype(
        jnp.float32
    )
    lhs_t = jnp.concatenate([xnt_ref[...], oh], axis=0)
    out_ref[...] = jax.lax.dot_general(
        lhs_t, m_ref[...],
        dimension_numbers=(((0,), (0,)), ((), ())),
        preferred_element_type=jnp.float32,
    )


def kernel(edge_attr_cat, edge_attr_num, emb_acc, emb_trans, emb_season,
           W_num, b_num, W_out, b_out):
    f32 = jnp.float32
    z = lambda n: jnp.zeros((n, 128), dtype=f32)
    a0 = jnp.concatenate([z(16), emb_acc.astype(f32), z(62)], axis=0)
    a1 = jnp.concatenate([z(66), emb_trans.astype(f32), z(42)], axis=0)
    a2 = jnp.concatenate([z(86), emb_season.astype(f32), z(38)], axis=0)
    a3 = jnp.concatenate(
        [W_num.astype(f32), z(74), b_num.astype(f32)[None, :], z(37)], axis=0
    )
    astack = jnp.stack([a0, a1, a2, a3], axis=0)
    w_blocks = W_out.astype(f32).reshape(4, 128, 512)

    m = pl.pallas_call(
        _fold_kernel,
        out_shape=jax.ShapeDtypeStruct((128, 512), f32),
    )(astack, w_blocks, b_out.astype(f32)[None, :])

    idx = edge_attr_cat.astype(jnp.int32)
    i0 = idx[:, 0].reshape(NB, 1, BLOCK)
    i1 = idx[:, 1].reshape(NB, 1, BLOCK)
    i2 = idx[:, 2].reshape(NB, 1, BLOCK)
    xnum_t = edge_attr_num.astype(f32).T

    out = pl.pallas_call(
        _main_kernel,
        grid=(NB,),
        in_specs=[
            pl.BlockSpec((1, 1, BLOCK), lambda i: (i, 0, 0)),
            pl.BlockSpec((1, 1, BLOCK), lambda i: (i, 0, 0)),
            pl.BlockSpec((1, 1, BLOCK), lambda i: (i, 0, 0)),
            pl.BlockSpec((16, BLOCK), lambda i: (0, i)),
            pl.BlockSpec((128, 512), lambda i: (0, 0)),
        ],
        out_specs=pl.BlockSpec((BLOCK, 512), lambda i: (i, 0)),
        out_shape=jax.ShapeDtypeStruct((E, 512), f32),
    )(i0, i1, i2, xnum_t, m)
    return out
